# baseline (speedup 1.0000x reference)
import functools

import jax
import jax.numpy as jnp
from jax import lax
from jax.experimental import pallas as pl
from jax.experimental.pallas import tpu as pltpu
from jax.experimental.pallas import tpu_sc as plsc

_VOCAB = 100000
_D = 128
_B = 32
_M = 200
_TOK = 4
_NT = 50
_NC = 2
_NS = 16
_NW = _NC * _NS
_CHUNK = 100
_NCHUNK = _M // _CHUNK
_ROWS_SC = _NS * _M
_TV = 2048
_NV = (_VOCAB + _TV - 1) // _TV


def _sc_gather_body(idx_hbm, oidx_hbm, dec_hbm, c0, c1, c2, c3,
                    m_out, x_out, idx_v, oidx_v, rows_v, dec_v, xrows_v, acc):
    c = lax.axis_index("c")
    s = lax.axis_index("s")
    w = c * _NS + s
    tables = [c0, c1, c2, c3]
    pltpu.sync_copy(idx_hbm.at[w], idx_v)
    pltpu.sync_copy(oidx_hbm.at[s], oidx_v)
    for ht in range(4):
        for t in range(_TOK):
            for p in range(_NCHUNK):
                pltpu.sync_copy(tables[ht].at[idx_v.at[t * _NCHUNK + p]], rows_v)
                if t == 0:
                    pltpu.sync_copy(
                        rows_v,
                        acc.at[pl.ds(ht * _ROWS_SC + s * _M + p * _CHUNK, _CHUNK)])
                else:
                    pltpu.sync_copy(rows_v, acc.at[oidx_v.at[ht, p]], add=True)
        pltpu.sync_copy(acc.at[pl.ds(ht * _ROWS_SC + s * _M, _M)],
                        m_out.at[ht, pl.ds(w * _M, _M)])

    @pl.when(jnp.logical_and(c == 0, s == 0))
    def _():
        pltpu.sync_copy(dec_hbm, dec_v)
        pltpu.sync_copy(c0.at[dec_v.at[0]], xrows_v)
        pltpu.sync_copy(xrows_v, x_out)


def _sc_gather(idx, oidx, dec, C0, C1, C2, C3):
    mesh = plsc.VectorSubcoreMesh(core_axis_name="c", subcore_axis_name="s")
    fn = pl.kernel(
        _sc_gather_body,
        mesh=mesh,
        out_type=[jax.ShapeDtypeStruct((4, _B * _M, _D), jnp.float32),
                  jax.ShapeDtypeStruct((_B, _D), jnp.float32)],
        scratch_types=[pltpu.VMEM((2 * _TOK, _CHUNK), jnp.int32),
                       pltpu.VMEM((4, _NCHUNK, _CHUNK), jnp.int32),
                       pltpu.VMEM((_CHUNK, _D), jnp.float32),
                       pltpu.VMEM((1, _B), jnp.int32),
                       pltpu.VMEM((_B, _D), jnp.float32),
                       pltpu.VMEM_SHARED((4 * _ROWS_SC, _D), jnp.float32)],
    )
    return fn(idx, oidx, dec, C0, C1, C2, C3)


def _dense_body(x_ref, h0_ref, roots_ref, bias_ref, m_ref,
                wq_ref, wk_ref, wv_ref, wih_ref, whh_ref, bih_ref, bhh_ref,
                pptr_ref, cur_ref, uo_ref):
    h0 = h0_ref[...]
    q = lax.dot(h0, wq_ref[...], preferred_element_type=jnp.float32)
    roots = roots_ref[...]
    roots2 = roots.reshape(_B * _NT, _D)
    rk = lax.dot(roots2, wk_ref[...], preferred_element_type=jnp.float32)
    rv = lax.dot(roots2, wv_ref[...], preferred_element_type=jnp.float32)
    rk = rk.reshape(_B, _NT, _D)
    rv = rv.reshape(_B, _NT, _D)
    rk_b = rk.astype(jnp.bfloat16).astype(jnp.float32)
    q_b = q.astype(jnp.bfloat16).astype(jnp.float32)
    al = jnp.sum(rk_b * q_b[:, None, :], axis=2) + bias_ref[...]
    aw = jax.nn.softmax(al, axis=1)
    kb = jnp.sum(aw[:, :, None] * rv, axis=1)

    x = x_ref[...]
    gi = lax.dot(x, wih_ref[...], preferred_element_type=jnp.float32) + bih_ref[...]
    gh = lax.dot(h0, whh_ref[...], preferred_element_type=jnp.float32) + bhh_ref[...]
    r = jax.nn.sigmoid(gi[:, 0:_D] + gh[:, 0:_D])
    z = jax.nn.sigmoid(gi[:, _D:2 * _D] + gh[:, _D:2 * _D])
    n = jnp.tanh(gi[:, 2 * _D:3 * _D] + r * gh[:, 2 * _D:3 * _D])
    hidden = (1.0 - z) * n + z * h0
    cur = hidden + kb
    cur_ref[...] = cur

    u = cur
    for hop in range(3):
        m_a = m_ref[hop]
        logits = jnp.sum(m_a * u[:, None, :], axis=2)
        prob = jax.nn.softmax(logits, axis=1)
        m_c = m_ref[hop + 1]
        o_k = jnp.sum(m_c * prob[:, :, None], axis=1)
        if hop == 0:
            uo_ref[:, 0:_D] = u
            uo_ref[:, _D:2 * _D] = o_k
        u = u + o_k
        if hop == 2:
            pptr_ref[...] = logits


def _dense(x, h0, roots, bias, m4, Wq, Wk, Wv, W_ih, W_hh, b_ih2, b_hh2):
    return pl.pallas_call(
        _dense_body,
        out_shape=[jax.ShapeDtypeStruct((_B, _M), jnp.float32),
                   jax.ShapeDtypeStruct((_B, _D), jnp.float32),
                   jax.ShapeDtypeStruct((_B, 2 * _D), jnp.float32)],
    )(x, h0, roots, bias, m4, Wq, Wk, Wv, W_ih, W_hh, b_ih2, b_hh2)


def _vocab_body(uo_ref, w_ref, b_ref, out_ref, logit_ref, m_ref, s_ref):
    i = pl.program_id(0)

    @pl.when(i == 0)
    def _():
        m_ref[...] = jnp.full((_B, 128), -3e38, jnp.float32)
        s_ref[...] = jnp.zeros((_B, 128), jnp.float32)

    @pl.when(i < _NV)
    def _():
        uo = uo_ref[...].astype(jnp.bfloat16)
        w = w_ref[...].astype(jnp.bfloat16)
        logits = lax.dot(uo, w, preferred_element_type=jnp.float32) + b_ref[...]
        col = i * _TV + lax.broadcasted_iota(jnp.int32, (_B, _TV), 1)
        logits = jnp.where(col < _VOCAB, logits, -1e30)
        logit_ref[:, pl.ds(i * _TV, _TV)] = logits
        t_max = jnp.max(logits, axis=1, keepdims=True)
        m_old = m_ref[...]
        m_new = jnp.maximum(m_old, t_max)
        ssum = jnp.sum(jnp.exp(logits - m_new[:, :1]), axis=1, keepdims=True)
        s_ref[...] = s_ref[...] * jnp.exp(m_old - m_new) + ssum
        m_ref[...] = m_new

    @pl.when(i >= _NV)
    def _():
        j = i - _NV
        lg = logit_ref[:, pl.ds(j * _TV, _TV)]
        out_ref[...] = jnp.exp(lg - m_ref[:, :1]) / s_ref[:, :1]


def _vocab(uo, W1, b12):
    return pl.pallas_call(
        _vocab_body,
        grid=(2 * _NV,),
        in_specs=[
            pl.BlockSpec((_B, 2 * _D), lambda i: (0, 0)),
            pl.BlockSpec((2 * _D, _TV), lambda i: (0, lax.min(i, _NV - 1))),
            pl.BlockSpec((1, _TV), lambda i: (0, lax.min(i, _NV - 1))),
        ],
        out_specs=pl.BlockSpec((_B, _TV), lambda i: (0, lax.max(i - _NV, 0))),
        out_shape=jax.ShapeDtypeStruct((_B, _VOCAB), jnp.float32),
        scratch_shapes=[pltpu.VMEM((_B, _NV * _TV), jnp.float32),
                        pltpu.VMEM((_B, 128), jnp.float32),
                        pltpu.VMEM((_B, 128), jnp.float32)],
    )(uo, W1, b12)


def kernel(decoder_input, story, hidden_states, roots_embed, attention_bias,
           global_index, C0, C1, C2, C3, Wq, Wk, Wv, W1, b1,
           W_ih, W_hh, b_ih, b_hh):
    story = story.astype(jnp.int32)
    dec = decoder_input.astype(jnp.int32).reshape(1, _B)
    idx = story.transpose(0, 2, 1).reshape(_NW, _TOK * _NCHUNK, _CHUNK)
    oidx = (jnp.arange(4, dtype=jnp.int32)[None, :, None, None] * _ROWS_SC
            + jnp.arange(_NS, dtype=jnp.int32)[:, None, None, None] * _M
            + jnp.arange(_NCHUNK, dtype=jnp.int32)[None, None, :, None] * _CHUNK
            + jnp.arange(_CHUNK, dtype=jnp.int32)[None, None, None, :])
    m_flat, x = _sc_gather(idx, oidx, dec, C0, C1, C2, C3)
    m4 = m_flat.reshape(4, _B, _M, _D)
    h0 = hidden_states[0]
    bias = attention_bias[:, :, 0]
    pptr, cur, uo = _dense(x, h0, roots_embed, bias, m4, Wq, Wk, Wv,
                           W_ih, W_hh, b_ih.reshape(1, -1), b_hh.reshape(1, -1))
    pvocab = _vocab(uo, W1, b1.reshape(1, -1))
    return (pptr, pvocab, cur[None])
